# baseline (speedup 1.0000x reference)
import functools

import jax
import jax.numpy as jnp
from jax import lax
from jax.experimental import pallas as pl
from jax.experimental.pallas import tpu as pltpu
from jax.experimental.pallas import tpu_sc as plsc

_VOCAB = 1000
_NDB = 100000
_D = 512
_BN = 2048
_NBLK = (_NDB + _BN - 1) // _BN


def _main_body(x_ref, w_ref, f_ref, dist_ref, closest_ref,
               feats_scr, minv_scr, mini_scr):
    i = pl.program_id(0)

    @pl.when(i == 0)
    def _init():
        y = lax.dot(x_ref[...], w_ref[...],
                    precision=lax.Precision.HIGHEST,
                    preferred_element_type=jnp.float32)
        feats_scr[...] = jnp.max(y.reshape(64, 49, _D), axis=1)
        minv_scr[...] = jnp.full((64, 1), jnp.inf, jnp.float32)
        mini_scr[...] = jnp.zeros((64, 1), jnp.int32)

    feats = feats_scr[...]
    f = f_ref[...]
    dots = lax.dot_general(feats, f, (((1,), (1,)), ((), ())),
                           precision=lax.Precision.HIGHEST,
                           preferred_element_type=jnp.float32)
    b2 = lax.dot_general(jnp.ones((1, _D), jnp.float32), f * f,
                         (((1,), (1,)), ((), ())),
                         precision=lax.Precision.HIGHEST,
                         preferred_element_type=jnp.float32)
    a2 = jnp.sum(feats * feats, axis=1, keepdims=True)
    sq = a2 + b2 - 2.0 * dots
    col = i * _BN + lax.broadcasted_iota(jnp.int32, (64, _BN), 1)
    sq = jnp.where(col < _NDB, sq, jnp.inf)
    dist_ref[...] = jnp.sqrt(jnp.maximum(sq, 0.0))
    blk_min = jnp.min(sq, axis=1, keepdims=True)
    blk_arg = jnp.min(jnp.where(sq == blk_min, col, _NDB),
                      axis=1, keepdims=True)
    better = blk_min < minv_scr[...]
    mini_scr[...] = jnp.where(better, blk_arg, mini_scr[...])
    minv_scr[...] = jnp.where(better, blk_min, minv_scr[...])

    @pl.when(i == pl.num_programs(0) - 1)
    def _fin():
        closest_ref[...] = mini_scr[...]


_main_call = pl.pallas_call(
    _main_body,
    grid=(_NBLK,),
    in_specs=[
        pl.BlockSpec((3136, _D), lambda i: (0, 0)),
        pl.BlockSpec((_D, _D), lambda i: (0, 0)),
        pl.BlockSpec((_BN, _D), lambda i: (i, 0)),
    ],
    out_specs=[
        pl.BlockSpec((64, _BN), lambda i: (0, i)),
        pl.BlockSpec((64, 1), lambda i: (0, 0)),
    ],
    out_shape=[
        jax.ShapeDtypeStruct((64, _NDB), jnp.float32),
        jax.ShapeDtypeStruct((64, 1), jnp.int32),
    ],
    scratch_shapes=[
        pltpu.VMEM((64, _D), jnp.float32),
        pltpu.VMEM((64, 1), jnp.float32),
        pltpu.VMEM((64, 1), jnp.int32),
    ],
    compiler_params=pltpu.CompilerParams(
        dimension_semantics=("arbitrary",)),
)


_NW = 80
_GROW = 78125


@functools.cache
def _make_sc_gather():
    @functools.partial(
        pl.kernel,
        mesh=plsc.VectorSubcoreMesh(core_axis_name="c", subcore_axis_name="s"),
        out_type=jax.ShapeDtypeStruct((_NW, 64), jnp.int32),
        scratch_types=[
            pltpu.VMEM((64,), jnp.int32),
            pltpu.VMEM((128,), jnp.int32),
            pltpu.VMEM((128, 128), jnp.int32),
            pltpu.VMEM((_NW, 64), jnp.int32),
            pltpu.SemaphoreType.DMA,
        ],
        compiler_params=pltpu.CompilerParams(use_tc_tiling_on_sc=False,
                                             needs_layout_passes=False),
    )
    def _sc_gather(idx_hbm, table_hbm, out_hbm, idx_v, gidx_v, rows_v,
                   out_v, sem):
        cid = lax.axis_index("c")
        sid = lax.axis_index("s")

        @pl.when(jnp.logical_and(cid == 0, sid == 0))
        def _():
            pltpu.sync_copy(idx_hbm, idx_v)
            for c in range(4):
                r = idx_v[pl.ds(c * 16, 16)]
                g0 = (r * 100) >> 7
                gidx_v[pl.ds(c * 16, 16)] = g0
                gidx_v[pl.ds(64 + c * 16, 16)] = jnp.minimum(g0 + 1, _GROW - 1)
            pltpu.async_copy(table_hbm.at[gidx_v], rows_v, sem).wait()
            for c in range(4):
                r = idx_v[pl.ds(c * 16, 16)]
                off = (r * 100) & 127
                b = c * 16 + lax.iota(jnp.int32, 16)
                for w in range(_NW):
                    ow = off + w
                    row = jnp.where(ow < 128, b, b + 64)
                    out_v[w, pl.ds(c * 16, 16)] = plsc.load_gather(
                        rows_v, [row, ow & 127])
            pltpu.sync_copy(out_v, out_hbm)

    return _sc_gather


def _oh_body(ids_ref, oh_ref):
    ids = ids_ref[...]
    iota = lax.broadcasted_iota(jnp.int32, (8, 80, _VOCAB), 2)
    oh_ref[...] = (ids[:, :, None] == iota).astype(jnp.float32)


_oh_call = pl.pallas_call(
    _oh_body,
    grid=(8,),
    in_specs=[pl.BlockSpec((8, 80), lambda i: (i, 0))],
    out_specs=pl.BlockSpec((8, 80, _VOCAB), lambda i: (i, 0, 0)),
    out_shape=jax.ShapeDtypeStruct((64, 80, _VOCAB), jnp.float32),
)


def kernel(images, reports, W_cnn, all_features, all_reports):
    b = images.shape[0]
    x = images.reshape(b, _D, 49).transpose(0, 2, 1).reshape(b * 49, _D)
    dist, closest = _main_call(x, W_cnn, all_features)
    sel = _make_sc_gather()(closest.reshape(b), all_reports.reshape(_GROW, 128))
    ids = sel.T[:, : reports.shape[1]]
    oh = _oh_call(ids)
    return (oh, dist)


# speedup vs baseline: 1.0002x; 1.0002x over previous
import functools

import jax
import jax.numpy as jnp
from jax import lax
from jax.experimental import pallas as pl
from jax.experimental.pallas import tpu as pltpu
from jax.experimental.pallas import tpu_sc as plsc

_VOCAB = 1000
_NDB = 100000
_D = 512
_BN = 2048
_NBLK = (_NDB + _BN - 1) // _BN


def _main_body(x_ref, w_ref, f_ref, dist_ref, closest_ref,
               feats_scr, minv_scr, mini_scr):
    i = pl.program_id(0)

    @pl.when(i == 0)
    def _init():
        y = lax.dot(x_ref[...], w_ref[...],
                    precision=lax.Precision.HIGHEST,
                    preferred_element_type=jnp.float32)
        feats_scr[...] = jnp.max(y.reshape(64, 49, _D), axis=1)
        minv_scr[...] = jnp.full((64, 1), jnp.inf, jnp.float32)
        mini_scr[...] = jnp.zeros((64, 1), jnp.int32)

    feats = feats_scr[...]
    f = f_ref[...]
    dots = lax.dot_general(feats, f, (((1,), (1,)), ((), ())),
                           precision=lax.Precision.HIGHEST,
                           preferred_element_type=jnp.float32)
    b2 = lax.dot_general(jnp.ones((1, _D), jnp.float32), f * f,
                         (((1,), (1,)), ((), ())),
                         precision=lax.Precision.HIGHEST,
                         preferred_element_type=jnp.float32)
    a2 = jnp.sum(feats * feats, axis=1, keepdims=True)
    sq = a2 + b2 - 2.0 * dots
    col = i * _BN + lax.broadcasted_iota(jnp.int32, (64, _BN), 1)
    sq = jnp.where(col < _NDB, sq, jnp.inf)
    dist_ref[...] = jnp.sqrt(jnp.maximum(sq, 0.0))
    blk_min = jnp.min(sq, axis=1, keepdims=True)
    blk_arg = jnp.min(jnp.where(sq == blk_min, col, _NDB),
                      axis=1, keepdims=True)
    better = blk_min < minv_scr[...]
    mini_scr[...] = jnp.where(better, blk_arg, mini_scr[...])
    minv_scr[...] = jnp.where(better, blk_min, minv_scr[...])

    @pl.when(i == pl.num_programs(0) - 1)
    def _fin():
        closest_ref[...] = mini_scr[...]


_main_call = pl.pallas_call(
    _main_body,
    grid=(_NBLK,),
    in_specs=[
        pl.BlockSpec((3136, _D), lambda i: (0, 0)),
        pl.BlockSpec((_D, _D), lambda i: (0, 0)),
        pl.BlockSpec((_BN, _D), lambda i: (i, 0)),
    ],
    out_specs=[
        pl.BlockSpec((64, _BN), lambda i: (0, i)),
        pl.BlockSpec((64, 1), lambda i: (0, 0)),
    ],
    out_shape=[
        jax.ShapeDtypeStruct((64, _NDB), jnp.float32),
        jax.ShapeDtypeStruct((64, 1), jnp.int32),
    ],
    scratch_shapes=[
        pltpu.VMEM((64, _D), jnp.float32),
        pltpu.VMEM((64, 1), jnp.float32),
        pltpu.VMEM((64, 1), jnp.int32),
    ],
    compiler_params=pltpu.CompilerParams(
        dimension_semantics=("arbitrary",)),
)


_NW = 80
_GROW = 78125


@functools.cache
def _make_sc_gather():
    @functools.partial(
        pl.kernel,
        mesh=plsc.VectorSubcoreMesh(core_axis_name="c", subcore_axis_name="s"),
        out_type=jax.ShapeDtypeStruct((_NW, 64), jnp.int32),
        scratch_types=[
            pltpu.VMEM((64,), jnp.int32),
            pltpu.VMEM((128,), jnp.int32),
            pltpu.VMEM((128, 128), jnp.int32),
            pltpu.VMEM((_NW, 64), jnp.int32),
            pltpu.SemaphoreType.DMA,
        ],
        compiler_params=pltpu.CompilerParams(use_tc_tiling_on_sc=True,
                                             needs_layout_passes=False),
    )
    def _sc_gather(idx_hbm, table_hbm, out_hbm, idx_v, gidx_v, rows_v,
                   out_v, sem):
        cid = lax.axis_index("c")
        sid = lax.axis_index("s")

        @pl.when(jnp.logical_and(cid == 0, sid == 0))
        def _():
            pltpu.sync_copy(idx_hbm, idx_v)
            for c in range(4):
                r = idx_v[pl.ds(c * 16, 16)]
                g0 = (r * 100) >> 7
                gidx_v[pl.ds(c * 16, 16)] = g0
                gidx_v[pl.ds(64 + c * 16, 16)] = jnp.minimum(g0 + 1, _GROW - 1)
            pltpu.async_copy(table_hbm.at[gidx_v], rows_v, sem).wait()
            for c in range(4):
                r = idx_v[pl.ds(c * 16, 16)]
                off = (r * 100) & 127
                b = c * 16 + lax.iota(jnp.int32, 16)
                for w in range(_NW):
                    ow = off + w
                    row = jnp.where(ow < 128, b, b + 64)
                    out_v[w, pl.ds(c * 16, 16)] = plsc.load_gather(
                        rows_v, [row, ow & 127])
            pltpu.sync_copy(out_v, out_hbm)

    return _sc_gather


def _oh_body(ids_ref, oh_ref):
    ids = ids_ref[...]
    iota = lax.broadcasted_iota(jnp.int32, (8, 80, _VOCAB), 2)
    oh_ref[...] = (ids[:, :, None] == iota).astype(jnp.float32)


_oh_call = pl.pallas_call(
    _oh_body,
    grid=(8,),
    in_specs=[pl.BlockSpec((8, 80), lambda i: (i, 0))],
    out_specs=pl.BlockSpec((8, 80, _VOCAB), lambda i: (i, 0, 0)),
    out_shape=jax.ShapeDtypeStruct((64, 80, _VOCAB), jnp.float32),
)


def kernel(images, reports, W_cnn, all_features, all_reports):
    b = images.shape[0]
    x = images.reshape(b, _D, 49).transpose(0, 2, 1).reshape(b * 49, _D)
    dist, closest = _main_call(x, W_cnn, all_features)
    sel = _make_sc_gather()(closest.reshape(b), all_reports.reshape(_GROW, 128))
    ids = sel.T[:, : reports.shape[1]]
    oh = _oh_call(ids)
    return (oh, dist)


# speedup vs baseline: 1.0967x; 1.0965x over previous
import functools

import jax
import jax.numpy as jnp
from jax import lax
from jax.experimental import pallas as pl
from jax.experimental.pallas import tpu as pltpu
from jax.experimental.pallas import tpu_sc as plsc

_VOCAB = 1000
_NDB = 100000
_D = 512
_BN = 2048
_NBLK = (_NDB + _BN - 1) // _BN


def _main_body(x_ref, w_ref, f_ref, dist_ref, closest_ref,
               feats_scr, minv_scr, mini_scr):
    i = pl.program_id(0)

    @pl.when(i == 0)
    def _init():
        y = lax.dot(x_ref[...], w_ref[...],
                    precision=lax.Precision.HIGHEST,
                    preferred_element_type=jnp.float32)
        feats_scr[...] = jnp.max(y.reshape(64, 49, _D), axis=1)
        minv_scr[...] = jnp.full((64, 1), jnp.inf, jnp.float32)
        mini_scr[...] = jnp.zeros((64, 1), jnp.int32)

    feats = feats_scr[...]
    f = f_ref[...]
    dots = lax.dot_general(feats, f, (((1,), (1,)), ((), ())),
                           precision=lax.Precision.HIGHEST,
                           preferred_element_type=jnp.float32)
    b2 = lax.dot_general(jnp.ones((1, _D), jnp.float32), f * f,
                         (((1,), (1,)), ((), ())),
                         precision=lax.Precision.HIGHEST,
                         preferred_element_type=jnp.float32)
    a2 = jnp.sum(feats * feats, axis=1, keepdims=True)
    sq = a2 + b2 - 2.0 * dots
    col = i * _BN + lax.broadcasted_iota(jnp.int32, (64, _BN), 1)
    sq = jnp.where(col < _NDB, sq, jnp.inf)
    dist_ref[...] = jnp.sqrt(jnp.maximum(sq, 0.0))
    blk_min = jnp.min(sq, axis=1, keepdims=True)
    blk_arg = jnp.min(jnp.where(sq == blk_min, col, _NDB),
                      axis=1, keepdims=True)
    better = blk_min < minv_scr[...]
    mini_scr[...] = jnp.where(better, blk_arg, mini_scr[...])
    minv_scr[...] = jnp.where(better, blk_min, minv_scr[...])

    @pl.when(i == pl.num_programs(0) - 1)
    def _fin():
        closest_ref[...] = mini_scr[...]


_main_call = pl.pallas_call(
    _main_body,
    grid=(_NBLK,),
    in_specs=[
        pl.BlockSpec((3136, _D), lambda i: (0, 0)),
        pl.BlockSpec((_D, _D), lambda i: (0, 0)),
        pl.BlockSpec((_BN, _D), lambda i: (i, 0)),
    ],
    out_specs=[
        pl.BlockSpec((64, _BN), lambda i: (0, i)),
        pl.BlockSpec((64, 1), lambda i: (0, 0)),
    ],
    out_shape=[
        jax.ShapeDtypeStruct((64, _NDB), jnp.float32),
        jax.ShapeDtypeStruct((64, 1), jnp.int32),
    ],
    scratch_shapes=[
        pltpu.VMEM((64, _D), jnp.float32),
        pltpu.VMEM((64, 1), jnp.float32),
        pltpu.VMEM((64, 1), jnp.int32),
    ],
    compiler_params=pltpu.CompilerParams(
        dimension_semantics=("arbitrary",)),
)


@functools.cache
def _make_sc_gather():
    @functools.partial(
        pl.kernel,
        mesh=plsc.VectorSubcoreMesh(core_axis_name="c", subcore_axis_name="s"),
        out_type=jax.ShapeDtypeStruct((64, 100), jnp.int32),
        scratch_types=[
            pltpu.VMEM((64,), jnp.int32),
            pltpu.VMEM((64, 100), jnp.int32),
            pltpu.SemaphoreType.DMA,
        ],
        compiler_params=pltpu.CompilerParams(use_tc_tiling_on_sc=True,
                                             needs_layout_passes=False),
    )
    def _sc_gather(idx_hbm, table_hbm, out_hbm, idx_v, rows_v, sem):
        cid = lax.axis_index("c")
        sid = lax.axis_index("s")

        @pl.when(jnp.logical_and(cid == 0, sid == 0))
        def _():
            pltpu.sync_copy(idx_hbm, idx_v)

            def issue(b, carry):
                chunk = idx_v[pl.ds((b >> 4) << 4, 16)]
                mask = lax.iota(jnp.int32, 16) == (b & 15)
                r = jnp.sum(jnp.where(mask, chunk, 0))
                pltpu.async_copy(table_hbm.at[r], rows_v.at[b], sem)
                return carry

            lax.fori_loop(0, 64, issue, 0)
            pltpu.make_async_copy(table_hbm.at[pl.ds(0, 64)], rows_v,
                                  sem).wait()
            pltpu.sync_copy(rows_v, out_hbm)

    return _sc_gather


def _oh_body(ids_ref, oh_ref):
    ids = ids_ref[...]
    iota = lax.broadcasted_iota(jnp.int32, (8, 80, _VOCAB), 2)
    oh_ref[...] = (ids[:, :, None] == iota).astype(jnp.float32)


_oh_call = pl.pallas_call(
    _oh_body,
    grid=(8,),
    in_specs=[pl.BlockSpec((8, 80), lambda i: (i, 0))],
    out_specs=pl.BlockSpec((8, 80, _VOCAB), lambda i: (i, 0, 0)),
    out_shape=jax.ShapeDtypeStruct((64, 80, _VOCAB), jnp.float32),
)


def kernel(images, reports, W_cnn, all_features, all_reports):
    b = images.shape[0]
    x = images.reshape(b, _D, 49).transpose(0, 2, 1).reshape(b * 49, _D)
    dist, closest = _main_call(x, W_cnn, all_features)
    sel = _make_sc_gather()(closest.reshape(b), all_reports)
    ids = sel[:, : reports.shape[1]]
    oh = _oh_call(ids)
    return (oh, dist)


# speedup vs baseline: 1.6592x; 1.5129x over previous
import functools

import jax
import jax.numpy as jnp
from jax import lax
from jax.experimental import pallas as pl
from jax.experimental.pallas import tpu as pltpu
from jax.experimental.pallas import tpu_sc as plsc

_VOCAB = 1000
_NDB = 100000
_D = 512
_BN = 4096
_NBLK = (_NDB + _BN - 1) // _BN


def _main_body(x_ref, w_ref, f_ref, dist_ref, closest_ref,
               feats_scr, minv_scr, mini_scr):
    i = pl.program_id(0)

    @pl.when(i == 0)
    def _init():
        y = lax.dot(x_ref[...], w_ref[...],
                    precision=lax.Precision.HIGHEST,
                    preferred_element_type=jnp.float32)
        feats_scr[...] = jnp.max(y.reshape(64, 49, _D), axis=1)
        minv_scr[...] = jnp.full((64, 1), jnp.inf, jnp.float32)
        mini_scr[...] = jnp.zeros((64, 1), jnp.int32)

    feats = feats_scr[...]
    f = f_ref[...]
    dots = lax.dot_general(feats, f, (((1,), (1,)), ((), ())),
                           precision=lax.Precision.HIGHEST,
                           preferred_element_type=jnp.float32)
    b2 = jnp.sum(f * f, axis=1)[None, :]
    a2 = jnp.sum(feats * feats, axis=1, keepdims=True)
    sq = a2 + b2 - 2.0 * dots
    col = i * _BN + lax.broadcasted_iota(jnp.int32, (64, _BN), 1)
    sq = jnp.where(col < _NDB, sq, jnp.inf)
    dist_ref[...] = jnp.sqrt(jnp.maximum(sq, 0.0))
    blk_min = jnp.min(sq, axis=1, keepdims=True)
    blk_arg = jnp.min(jnp.where(sq == blk_min, col, _NDB),
                      axis=1, keepdims=True)
    better = blk_min < minv_scr[...]
    mini_scr[...] = jnp.where(better, blk_arg, mini_scr[...])
    minv_scr[...] = jnp.where(better, blk_min, minv_scr[...])

    @pl.when(i == pl.num_programs(0) - 1)
    def _fin():
        closest_ref[...] = mini_scr[...]


_main_call = pl.pallas_call(
    _main_body,
    grid=(_NBLK,),
    in_specs=[
        pl.BlockSpec((3136, _D), lambda i: (0, 0)),
        pl.BlockSpec((_D, _D), lambda i: (0, 0)),
        pl.BlockSpec((_BN, _D), lambda i: (i, 0)),
    ],
    out_specs=[
        pl.BlockSpec((64, _BN), lambda i: (0, i)),
        pl.BlockSpec((64, 1), lambda i: (0, 0)),
    ],
    out_shape=[
        jax.ShapeDtypeStruct((64, _NDB), jnp.float32),
        jax.ShapeDtypeStruct((64, 1), jnp.int32),
    ],
    scratch_shapes=[
        pltpu.VMEM((64, _D), jnp.float32),
        pltpu.VMEM((64, 1), jnp.float32),
        pltpu.VMEM((64, 1), jnp.int32),
    ],
    compiler_params=pltpu.CompilerParams(
        dimension_semantics=("arbitrary",)),
)


@functools.cache
def _make_sc_gather():
    @functools.partial(
        pl.kernel,
        mesh=plsc.VectorSubcoreMesh(core_axis_name="c", subcore_axis_name="s"),
        out_type=jax.ShapeDtypeStruct((64, 100), jnp.int32),
        scratch_types=[
            pltpu.VMEM((64,), jnp.int32),
            pltpu.VMEM((64, 100), jnp.int32),
            pltpu.SemaphoreType.DMA,
        ],
        compiler_params=pltpu.CompilerParams(use_tc_tiling_on_sc=True,
                                             needs_layout_passes=False),
    )
    def _sc_gather(idx_hbm, table_hbm, out_hbm, idx_v, rows_v, sem):
        cid = lax.axis_index("c")
        sid = lax.axis_index("s")

        @pl.when(jnp.logical_and(cid == 0, sid == 0))
        def _():
            pltpu.sync_copy(idx_hbm, idx_v)

            def issue(b, carry):
                chunk = idx_v[pl.ds((b >> 4) << 4, 16)]
                mask = lax.iota(jnp.int32, 16) == (b & 15)
                r = jnp.sum(jnp.where(mask, chunk, 0))
                pltpu.async_copy(table_hbm.at[r], rows_v.at[b], sem)
                return carry

            lax.fori_loop(0, 64, issue, 0)
            pltpu.make_async_copy(table_hbm.at[pl.ds(0, 64)], rows_v,
                                  sem).wait()
            pltpu.sync_copy(rows_v, out_hbm)

    return _sc_gather


def _oh_body(ids_ref, oh_ref):
    ids = ids_ref[...]
    iota = lax.broadcasted_iota(jnp.int32, (8, 80, _VOCAB), 2)
    oh_ref[...] = (ids[:, :, None] == iota).astype(jnp.float32)


_oh_call = pl.pallas_call(
    _oh_body,
    grid=(8,),
    in_specs=[pl.BlockSpec((8, 80), lambda i: (i, 0))],
    out_specs=pl.BlockSpec((8, 80, _VOCAB), lambda i: (i, 0, 0)),
    out_shape=jax.ShapeDtypeStruct((64, 80, _VOCAB), jnp.float32),
)


def kernel(images, reports, W_cnn, all_features, all_reports):
    b = images.shape[0]
    x = images.reshape(b, _D, 49).transpose(0, 2, 1).reshape(b * 49, _D)
    dist, closest = _main_call(x, W_cnn, all_features)
    sel = _make_sc_gather()(closest.reshape(b), all_reports)
    ids = sel[:, : reports.shape[1]]
    oh = _oh_call(ids)
    return (oh, dist)
